# baseline (speedup 1.0000x reference)
import functools
import jax
import jax.numpy as jnp
from jax.experimental import pallas as pl

_N = 150528
_P = 50176
_K = 12544
_HI0 = 0x3F800000
_BIG = 1 << 30


def _search_body(g_ref, m_ref, *, k, r=16):
    g = g_ref[...]
    keys = jax.lax.bitcast_convert_type(g, jnp.int32)
    kf = jnp.float32(k)

    def count_ge(thr):
        m = (keys >= thr).astype(jnp.float32)
        parts = [
            jnp.sum(m[:, j * (_N // 16):(j + 1) * (_N // 16)], axis=1, keepdims=True)
            for j in range(16)
        ]
        return sum(parts)

    def vcond(carry):
        i, lo, hi, cnt_lo = carry
        return (i < 31) & jnp.any((cnt_lo != kf) & (hi - lo > 1))

    def vstep(carry):
        i, lo, hi, cnt_lo = carry
        mid = lo + (hi - lo) // 2
        c = count_ge(mid)
        big = c >= kf
        return (i + 1, jnp.where(big, mid, lo), jnp.where(big, hi, mid),
                jnp.where(big, c, cnt_lo))

    _, lo, hi, cnt_lo = jax.lax.while_loop(
        vcond, vstep,
        (jnp.int32(0), jnp.zeros((r, 1), jnp.int32),
         jnp.full((r, 1), _HI0, jnp.int32),
         jnp.full((r, 1), float(_N), jnp.float32)))
    exact = cnt_lo == kf
    t = jnp.where(exact, lo - 1, lo)

    idx = jax.lax.broadcasted_iota(jnp.int32, (r, _N), 1)

    def do_idx_search(_):
        need = kf - jnp.sum((keys > t).astype(jnp.float32), axis=1, keepdims=True)
        w = jnp.where(keys == t, idx, _BIG)

        def istep(_, carry):
            lo2, hi2 = carry
            mid = lo2 + (hi2 - lo2) // 2
            cnt = jnp.sum((w <= mid).astype(jnp.float32), axis=1, keepdims=True)
            ge = cnt >= need
            return jnp.where(ge, lo2, mid + 1), jnp.where(ge, mid, hi2)

        cs, _ = jax.lax.fori_loop(
            0, 18, istep,
            (jnp.zeros((r, 1), jnp.int32), jnp.full((r, 1), _N - 1, jnp.int32)),
            unroll=False)
        return cs

    cstar = jax.lax.cond(
        jnp.any(~exact), do_idx_search,
        lambda _: jnp.full((r, 1), -1, jnp.int32), None)
    cstar = jnp.where(exact, -1, cstar)

    sel = (keys > t) | ((keys == t) & (idx <= cstar))
    pix = sel[:, 0:_P] | sel[:, _P:2 * _P] | sel[:, 2 * _P:]
    keep = 1.0 - pix.astype(jnp.float32)
    m_ref[...] = keep.astype(jnp.bfloat16)


def _apply_body(d_ref, m_ref, o_ref):
    b = m_ref.shape[0]
    mf = m_ref[...].astype(jnp.float32)
    m = jnp.transpose(mf)
    m4 = m.reshape(8, 224, b)
    o_ref[...] = d_ref[...] * m4[:, None]


@jax.jit
def kernel(Data, line_grad):
    b = Data.shape[0]
    mask = pl.pallas_call(
        functools.partial(_search_body, k=_K),
        grid=(b // 16,),
        in_specs=[pl.BlockSpec((16, _N), lambda i: (i, 0))],
        out_specs=pl.BlockSpec((16, _P), lambda i: (i, 0)),
        out_shape=jax.ShapeDtypeStruct((b, _P), jnp.bfloat16),
    )(line_grad)

    dt = jnp.transpose(Data, (1, 3, 2, 0))
    ot = pl.pallas_call(
        _apply_body,
        grid=(28,),
        in_specs=[
            pl.BlockSpec((8, 3, 224, b), lambda h: (h, 0, 0, 0)),
            pl.BlockSpec((b, 1792), lambda h: (0, h)),
        ],
        out_specs=pl.BlockSpec((8, 3, 224, b), lambda h: (h, 0, 0, 0)),
        out_shape=jax.ShapeDtypeStruct((224, 3, 224, b), jnp.float32),
    )(dt, mask)
    return jnp.transpose(ot, (3, 0, 2, 1))


# speedup vs baseline: 1.3885x; 1.3885x over previous
import functools
import jax
import jax.numpy as jnp
from jax.experimental import pallas as pl

_N = 150528
_P = 50176
_K = 12544
_HI0 = 0x3F800000
_BIG = 1 << 30


def _search_body(g_ref, m_ref, *, k, r=16):
    g = g_ref[...]
    keys = jax.lax.bitcast_convert_type(g, jnp.int32)
    kf = jnp.float32(k)

    def count_ge(thr):
        m = (keys >= thr).astype(jnp.float32)
        parts = [
            jnp.sum(m[:, j * (_N // 8):(j + 1) * (_N // 8)], axis=1, keepdims=True)
            for j in range(8)
        ]
        return sum(parts)

    def vcond(carry):
        i, lo, hi, cnt_lo = carry
        return (i < 31) & jnp.any((cnt_lo != kf) & (hi - lo > 1))

    def vstep(carry):
        i, lo, hi, cnt_lo = carry
        mid = lo + (hi - lo) // 2
        c = count_ge(mid)
        big = c >= kf
        return (i + 1, jnp.where(big, mid, lo), jnp.where(big, hi, mid),
                jnp.where(big, c, cnt_lo))

    _, lo, hi, cnt_lo = jax.lax.while_loop(
        vcond, vstep,
        (jnp.int32(0), jnp.zeros((r, 1), jnp.int32),
         jnp.full((r, 1), _HI0, jnp.int32),
         jnp.full((r, 1), float(_N), jnp.float32)))
    exact = cnt_lo == kf
    t = jnp.where(exact, lo - 1, lo)

    idx = jax.lax.broadcasted_iota(jnp.int32, (r, _N), 1)

    def do_idx_search(_):
        need = kf - jnp.sum((keys > t).astype(jnp.float32), axis=1, keepdims=True)
        w = jnp.where(keys == t, idx, _BIG)

        def istep(_, carry):
            lo2, hi2 = carry
            mid = lo2 + (hi2 - lo2) // 2
            cnt = jnp.sum((w <= mid).astype(jnp.float32), axis=1, keepdims=True)
            ge = cnt >= need
            return jnp.where(ge, lo2, mid + 1), jnp.where(ge, mid, hi2)

        cs, _ = jax.lax.fori_loop(
            0, 18, istep,
            (jnp.zeros((r, 1), jnp.int32), jnp.full((r, 1), _N - 1, jnp.int32)),
            unroll=False)
        return cs

    cstar = jax.lax.cond(
        jnp.any(~exact), do_idx_search,
        lambda _: jnp.full((r, 1), -1, jnp.int32), None)
    cstar = jnp.where(exact, -1, cstar)

    sel = (keys > t) | ((keys == t) & (idx <= cstar))
    pix = sel[:, 0:_P] | sel[:, _P:2 * _P] | sel[:, 2 * _P:]
    keep = 1.0 - pix.astype(jnp.float32)
    m_ref[...] = keep.astype(jnp.bfloat16)


def _apply_body(d_ref, m_ref, o_ref):
    b = m_ref.shape[0]
    mf = m_ref[...].astype(jnp.float32)
    m = jnp.transpose(mf)
    m4 = m.reshape(8, 224, b)
    o_ref[...] = d_ref[...] * m4[:, None]


@jax.jit
def kernel(Data, line_grad):
    b = Data.shape[0]
    mask = pl.pallas_call(
        functools.partial(_search_body, k=_K),
        grid=(b // 16,),
        in_specs=[pl.BlockSpec((16, _N), lambda i: (i, 0))],
        out_specs=pl.BlockSpec((16, _P), lambda i: (i, 0)),
        out_shape=jax.ShapeDtypeStruct((b, _P), jnp.bfloat16),
    )(line_grad)

    dt = jnp.transpose(Data, (1, 3, 2, 0))
    ot = pl.pallas_call(
        _apply_body,
        grid=(28,),
        in_specs=[
            pl.BlockSpec((8, 3, 224, b), lambda h: (h, 0, 0, 0)),
            pl.BlockSpec((b, 1792), lambda h: (0, h)),
        ],
        out_specs=pl.BlockSpec((8, 3, 224, b), lambda h: (h, 0, 0, 0)),
        out_shape=jax.ShapeDtypeStruct((224, 3, 224, b), jnp.float32),
    )(dt, mask)
    return jnp.transpose(ot, (3, 0, 2, 1))


# speedup vs baseline: 1.8536x; 1.3350x over previous
import functools
import jax
import jax.numpy as jnp
from jax.experimental import pallas as pl

_N = 150528
_P = 50176
_K = 12544
_HI0 = 0x3F800000
_BIG = 1 << 30


def _search_body(g_ref, m_ref, *, k, r=16):
    g = g_ref[...]
    keys = jax.lax.bitcast_convert_type(g, jnp.int32)
    kf = jnp.float32(k)

    def count_ge(thr):
        m = (keys >= thr).astype(jnp.float32)
        parts = [
            jnp.sum(m[:, j * (_N // 8):(j + 1) * (_N // 8)], axis=1, keepdims=True)
            for j in range(8)
        ]
        return sum(parts)

    def vcond(carry):
        i, lo, hi, cnt_lo = carry
        return (i < 31) & jnp.any((cnt_lo != kf) & (hi - lo > 1))

    def vstep(carry):
        i, lo, hi, cnt_lo = carry
        mid = lo + (hi - lo) // 2
        c = count_ge(mid)
        big = c >= kf
        return (i + 1, jnp.where(big, mid, lo), jnp.where(big, hi, mid),
                jnp.where(big, c, cnt_lo))

    g_lo = jax.lax.bitcast_convert_type(jnp.float32(0.910), jnp.int32)
    g_hi = jax.lax.bitcast_convert_type(jnp.float32(0.9235), jnp.int32)
    c_gl = count_ge(g_lo)
    c_gh = count_ge(g_hi)
    ok_lo = c_gl >= kf
    lo0 = jnp.where(ok_lo, g_lo, 0)
    cl0 = jnp.where(ok_lo, c_gl, jnp.float32(_N))
    hi0 = jnp.where(c_gh < kf, g_hi, _HI0)

    _, lo, hi, cnt_lo = jax.lax.while_loop(
        vcond, vstep, (jnp.int32(0), lo0, hi0, cl0))
    exact = cnt_lo == kf
    t = jnp.where(exact, lo - 1, lo)

    idx = jax.lax.broadcasted_iota(jnp.int32, (r, _N), 1)

    def do_idx_search(_):
        need = kf - jnp.sum((keys > t).astype(jnp.float32), axis=1, keepdims=True)
        w = jnp.where(keys == t, idx, _BIG)

        def istep(_, carry):
            lo2, hi2 = carry
            mid = lo2 + (hi2 - lo2) // 2
            cnt = jnp.sum((w <= mid).astype(jnp.float32), axis=1, keepdims=True)
            ge = cnt >= need
            return jnp.where(ge, lo2, mid + 1), jnp.where(ge, mid, hi2)

        cs, _ = jax.lax.fori_loop(
            0, 18, istep,
            (jnp.zeros((r, 1), jnp.int32), jnp.full((r, 1), _N - 1, jnp.int32)),
            unroll=False)
        return cs

    cstar = jax.lax.cond(
        jnp.any(~exact), do_idx_search,
        lambda _: jnp.full((r, 1), -1, jnp.int32), None)
    cstar = jnp.where(exact, -1, cstar)

    sel = (keys > t) | ((keys == t) & (idx <= cstar))
    pix = sel[:, 0:_P] | sel[:, _P:2 * _P] | sel[:, 2 * _P:]
    keep = 1.0 - pix.astype(jnp.float32)
    m_ref[...] = keep.astype(jnp.bfloat16)


def _apply_body(d_ref, m_ref, o_ref):
    b = m_ref.shape[0]
    mf = m_ref[...].astype(jnp.float32)
    m = jnp.transpose(mf)
    m4 = m.reshape(8, 224, b)
    o_ref[...] = d_ref[...] * m4[:, None]


@jax.jit
def kernel(Data, line_grad):
    b = Data.shape[0]
    mask = pl.pallas_call(
        functools.partial(_search_body, k=_K),
        grid=(b // 16,),
        in_specs=[pl.BlockSpec((16, _N), lambda i: (i, 0))],
        out_specs=pl.BlockSpec((16, _P), lambda i: (i, 0)),
        out_shape=jax.ShapeDtypeStruct((b, _P), jnp.bfloat16),
    )(line_grad)

    dt = jnp.transpose(Data, (1, 3, 2, 0))
    ot = pl.pallas_call(
        _apply_body,
        grid=(28,),
        in_specs=[
            pl.BlockSpec((8, 3, 224, b), lambda h: (h, 0, 0, 0)),
            pl.BlockSpec((b, 1792), lambda h: (0, h)),
        ],
        out_specs=pl.BlockSpec((8, 3, 224, b), lambda h: (h, 0, 0, 0)),
        out_shape=jax.ShapeDtypeStruct((224, 3, 224, b), jnp.float32),
    )(dt, mask)
    return jnp.transpose(ot, (3, 0, 2, 1))


# speedup vs baseline: 1.8974x; 1.0236x over previous
import functools
import jax
import jax.numpy as jnp
from jax.experimental import pallas as pl

_N = 150528
_P = 50176
_K = 12544
_HI0 = 0x3F800000
_BIG = 1 << 30


def _search_body(g_ref, m_ref, *, k, r=16):
    g = g_ref[...]
    keys = jax.lax.bitcast_convert_type(g, jnp.int32)
    kf = jnp.float32(k)

    def count_ge(thr):
        m = (keys >= thr).astype(jnp.float32)
        parts = [
            jnp.sum(m[:, j * (_N // 8):(j + 1) * (_N // 8)], axis=1, keepdims=True)
            for j in range(8)
        ]
        return sum(parts)

    def vcond(carry):
        i, lo, hi, cnt_lo = carry
        return (i < 31) & jnp.any((cnt_lo != kf) & (hi - lo > 1))

    def vstep(carry):
        i, lo, hi, cnt_lo = carry
        mid = lo + (hi - lo) // 2
        c = count_ge(mid)
        big = c >= kf
        return (i + 1, jnp.where(big, mid, lo), jnp.where(big, hi, mid),
                jnp.where(big, c, cnt_lo))

    g_lo = jax.lax.bitcast_convert_type(jnp.float32(0.9125), jnp.int32)
    g_hi = jax.lax.bitcast_convert_type(jnp.float32(0.9210), jnp.int32)
    c_gl = count_ge(g_lo)
    c_gh = count_ge(g_hi)
    ok_lo = c_gl >= kf
    lo0 = jnp.where(ok_lo, g_lo, 0)
    cl0 = jnp.where(ok_lo, c_gl, jnp.float32(_N))
    hi0 = jnp.where(c_gh < kf, g_hi, _HI0)

    _, lo, hi, cnt_lo = jax.lax.while_loop(
        vcond, vstep, (jnp.int32(0), lo0, hi0, cl0))
    exact = cnt_lo == kf
    t = jnp.where(exact, lo - 1, lo)

    idx = jax.lax.broadcasted_iota(jnp.int32, (r, _N), 1)

    def do_idx_search(_):
        need = kf - jnp.sum((keys > t).astype(jnp.float32), axis=1, keepdims=True)
        w = jnp.where(keys == t, idx, _BIG)

        def istep(_, carry):
            lo2, hi2 = carry
            mid = lo2 + (hi2 - lo2) // 2
            cnt = jnp.sum((w <= mid).astype(jnp.float32), axis=1, keepdims=True)
            ge = cnt >= need
            return jnp.where(ge, lo2, mid + 1), jnp.where(ge, mid, hi2)

        cs, _ = jax.lax.fori_loop(
            0, 18, istep,
            (jnp.zeros((r, 1), jnp.int32), jnp.full((r, 1), _N - 1, jnp.int32)),
            unroll=False)
        return cs

    cstar = jax.lax.cond(
        jnp.any(~exact), do_idx_search,
        lambda _: jnp.full((r, 1), -1, jnp.int32), None)
    cstar = jnp.where(exact, -1, cstar)

    sel = (keys > t) | ((keys == t) & (idx <= cstar))
    pix = sel[:, 0:_P] | sel[:, _P:2 * _P] | sel[:, 2 * _P:]
    keep = 1.0 - pix.astype(jnp.float32)
    m_ref[...] = keep.astype(jnp.bfloat16)


def _apply_body(d_ref, m_ref, o_ref):
    b = m_ref.shape[0]
    mf = m_ref[...].astype(jnp.float32)
    m = jnp.transpose(mf)
    m4 = m.reshape(8, 224, b)
    o_ref[...] = d_ref[...] * m4[:, None]


@jax.jit
def kernel(Data, line_grad):
    b = Data.shape[0]
    mask = pl.pallas_call(
        functools.partial(_search_body, k=_K),
        grid=(b // 16,),
        in_specs=[pl.BlockSpec((16, _N), lambda i: (i, 0))],
        out_specs=pl.BlockSpec((16, _P), lambda i: (i, 0)),
        out_shape=jax.ShapeDtypeStruct((b, _P), jnp.bfloat16),
    )(line_grad)

    dt = jnp.transpose(Data, (1, 3, 2, 0))
    ot = pl.pallas_call(
        _apply_body,
        grid=(28,),
        in_specs=[
            pl.BlockSpec((8, 3, 224, b), lambda h: (h, 0, 0, 0)),
            pl.BlockSpec((b, 1792), lambda h: (0, h)),
        ],
        out_specs=pl.BlockSpec((8, 3, 224, b), lambda h: (h, 0, 0, 0)),
        out_shape=jax.ShapeDtypeStruct((224, 3, 224, b), jnp.float32),
    )(dt, mask)
    return jnp.transpose(ot, (3, 0, 2, 1))
